# Initial kernel scaffold; baseline (speedup 1.0000x reference)
import functools

import jax
import jax.numpy as jnp
from jax import lax
from jax.experimental import pallas as pl
from jax.experimental.pallas import tpu as pltpu
from jax.experimental.pallas import tpu_sc as plsc

F32 = jnp.float32
I32 = jnp.int32

NC = 2
NS = 16
L = 16
CH = 128
ZR = 112


def _mesh():
    return plsc.VectorSubcoreMesh(core_axis_name="c", subcore_axis_name="s")


def _deg_body(npad, et, dst_hbm, ew_hbm, out_hbm, part, dstv, ewv):
    c = lax.axis_index("c")
    s = lax.axis_index("s")
    w = c * NS + s
    z16 = jnp.zeros((L,), F32)

    def zero(i, _):
        part[pl.ds(i * L, L)] = z16
        return 0

    lax.fori_loop(0, npad // L, zero, 0)

    base0 = w * et

    def chunk(t, _):
        b = base0 + t * CH
        pltpu.sync_copy(dst_hbm.at[pl.ds(b, CH)], dstv)
        pltpu.sync_copy(ew_hbm.at[pl.ds(b, CH)], ewv)
        for g in range(CH // L):
            d16 = dstv[pl.ds(g * L, L)]
            e16 = ewv[pl.ds(g * L, L)]
            plsc.addupdate_scatter(part, [d16], e16)
        return 0

    lax.fori_loop(0, et // CH, chunk, 0)
    pltpu.sync_copy(part, out_hbm.at[w])


def _make_deg(npad, epad):
    et = epad // (NC * NS)
    return pl.kernel(
        functools.partial(_deg_body, npad, et),
        out_type=jax.ShapeDtypeStruct((NC * NS, npad), F32),
        mesh=_mesh(),
        scratch_types=[
            pltpu.VMEM((npad,), F32),
            pltpu.VMEM((CH,), I32),
            pltpu.VMEM((CH,), F32),
        ],
    )


def _agg_body(npad, n_chunks, d, feature_split, src_hbm, dst_hbm, ew_hbm,
              v_hbm, out_hbm, srcv, dstv, idxv, ewv, rows, zbuf, acc, sem):
    c = lax.axis_index("c")
    s = lax.axis_index("s")
    tr = npad // NS
    nvec = d // L
    z16 = jnp.zeros((L,), F32)

    def zb(i, _):
        for k in range(nvec):
            zbuf[i, pl.ds(k * L, L)] = z16
        return 0

    lax.fori_loop(0, ZR, zb, 0)

    row0 = s * tr

    def za(j, _):
        pltpu.sync_copy(zbuf, acc.at[pl.ds(row0 + j * ZR, ZR)])
        return 0

    lax.fori_loop(0, tr // ZR, za, 0)
    plsc.subcore_barrier()

    if feature_split:
        base0 = s * (n_chunks * CH)
        goff = c * npad
    else:
        base0 = (c * NS + s) * (n_chunks * CH)
        goff = 0

    def chunk(t, _):
        b = base0 + t * CH
        pltpu.sync_copy(src_hbm.at[pl.ds(b, CH)], srcv)
        pltpu.sync_copy(dst_hbm.at[pl.ds(b, CH)], dstv)
        pltpu.sync_copy(ew_hbm.at[pl.ds(b, CH)], ewv)
        for g in range(CH // L):
            idxv[pl.ds(g * L, L)] = srcv[pl.ds(g * L, L)] + goff
        pltpu.async_copy(v_hbm.at[idxv], rows, sem).wait()

        def scale(g, _):
            for e in range(L):
                e2 = g * L + e
                sc = ewv[e2]
                for k in range(nvec):
                    rows[e2, pl.ds(k * L, L)] = rows[e2, pl.ds(k * L, L)] * sc
            return 0

        lax.fori_loop(0, CH // L, scale, 0)
        pltpu.sync_copy(rows, acc.at[dstv], add=True)
        return 0

    lax.fori_loop(0, n_chunks, chunk, 0)
    plsc.subcore_barrier()
    pltpu.sync_copy(acc.at[pl.ds(row0, tr)],
                    out_hbm.at[pl.ds(c * npad + row0, tr)])


def _make_agg(npad, epad, d, feature_split):
    n_chunks = epad // ((NS if feature_split else NC * NS) * CH)
    return pl.kernel(
        functools.partial(_agg_body, npad, n_chunks, d, feature_split),
        out_type=jax.ShapeDtypeStruct((NC * npad, d), F32),
        mesh=_mesh(),
        scratch_types=[
            pltpu.VMEM((CH,), I32),
            pltpu.VMEM((CH,), I32),
            pltpu.VMEM((CH,), I32),
            pltpu.VMEM((CH,), F32),
            pltpu.VMEM((CH, d), F32),
            pltpu.VMEM((ZR, d), F32),
            pltpu.VMEM_SHARED((npad, d), F32),
            pltpu.SemaphoreType.DMA,
        ],
    )


def _k2_body(parts_ref, x_ref, w1_ref, xs_ref, dis_ref):
    parts = parts_ref[...]
    ones = jnp.ones((parts.shape[0], 1), F32)
    deg = 1.0 + lax.dot_general(parts, ones, (((0,), (0,)), ((), ())),
                                preferred_element_type=F32)
    dis = lax.rsqrt(deg)
    xw = jnp.dot(x_ref[...], w1_ref[...], preferred_element_type=F32,
                 precision=lax.Precision.HIGHEST)
    xs_ref[...] = dis * xw
    dis_ref[...] = dis


def _k4_body(dis_ref, s0_ref, s1_ref, xs0_ref, xs1_ref, b1_ref, w2_ref,
             out_ref):
    dis = dis_ref[...]
    b1 = b1_ref[...]
    w2 = w2_ref[...]
    half = b1.shape[1] // 2
    h0 = jax.nn.relu(dis * (s0_ref[...] + xs0_ref[...]) + b1[:, :half])
    h1 = jax.nn.relu(dis * (s1_ref[...] + xs1_ref[...]) + b1[:, half:])
    hw2 = (jnp.dot(h0, w2[:half, :], preferred_element_type=F32,
                   precision=lax.Precision.HIGHEST)
           + jnp.dot(h1, w2[half:, :], preferred_element_type=F32,
                     precision=lax.Precision.HIGHEST))
    hs2 = dis * hw2
    pad = out_ref.shape[1] - hs2.shape[1]
    out_ref[...] = jnp.concatenate(
        [hs2, jnp.zeros((hs2.shape[0], pad), F32)], axis=1)


def _k6_body(d_out, dis_ref, s2a_ref, s2b_ref, hs_ref, b2_ref, out_ref):
    t = s2a_ref[...] + s2b_ref[...] + hs_ref[...]
    o = dis_ref[...] * t[:, :d_out] + b2_ref[...]
    m = jnp.max(o, axis=1, keepdims=True)
    lse = m + jnp.log(jnp.sum(jnp.exp(o - m), axis=1, keepdims=True))
    out_ref[...] = o - lse


def kernel(x, edge_index, edge_weight, W1, b1, W2, b2):
    n, d_in = x.shape
    e = edge_index.shape[1]
    d_hid = W1.shape[1]
    d_out = W2.shape[1]
    half = d_hid // 2
    d2p = 16

    r = 512
    npad = -(-n // r) * r
    nb = npad // r
    epad = -(-e // (NC * NS * CH)) * (NC * NS * CH)

    src = jnp.pad(edge_index[0], (0, epad - e))
    dst = jnp.pad(edge_index[1], (0, epad - e))
    ew = jnp.pad(edge_weight, (0, epad - e))
    xp = jnp.pad(x, ((0, npad - n), (0, 0)))

    parts = _make_deg(npad, epad)(dst, ew)

    xs_flat, dis = pl.pallas_call(
        _k2_body,
        grid=(NC, nb),
        in_specs=[
            pl.BlockSpec((NC * NS, r), lambda i, j: (0, j)),
            pl.BlockSpec((r, d_in), lambda i, j: (j, 0)),
            pl.BlockSpec((d_in, half), lambda i, j: (0, i)),
        ],
        out_specs=[
            pl.BlockSpec((r, half), lambda i, j: (i * (npad // r) + j, 0)),
            pl.BlockSpec((r, 1), lambda i, j: (j, 0)),
        ],
        out_shape=[
            jax.ShapeDtypeStruct((NC * npad, half), F32),
            jax.ShapeDtypeStruct((npad, 1), F32),
        ],
    )(parts, xp, W1)

    s1_flat = _make_agg(npad, epad, half, True)(src, dst, ew, xs_flat)

    hs2p = pl.pallas_call(
        _k4_body,
        grid=(nb,),
        in_specs=[
            pl.BlockSpec((r, 1), lambda j: (j, 0)),
            pl.BlockSpec((r, half), lambda j: (j, 0)),
            pl.BlockSpec((r, half), lambda j: (nb + j, 0)),
            pl.BlockSpec((r, half), lambda j: (j, 0)),
            pl.BlockSpec((r, half), lambda j: (nb + j, 0)),
            pl.BlockSpec((1, d_hid), lambda j: (0, 0)),
            pl.BlockSpec((d_hid, d_out), lambda j: (0, 0)),
        ],
        out_specs=pl.BlockSpec((r, d2p), lambda j: (j, 0)),
        out_shape=jax.ShapeDtypeStruct((npad, d2p), F32),
    )(dis, s1_flat, s1_flat, xs_flat, xs_flat, b1.reshape(1, d_hid), W2)

    s2_flat = _make_agg(npad, epad, d2p, False)(src, dst, ew, hs2p)

    out = pl.pallas_call(
        functools.partial(_k6_body, d_out),
        grid=(nb,),
        in_specs=[
            pl.BlockSpec((r, 1), lambda j: (j, 0)),
            pl.BlockSpec((r, d2p), lambda j: (j, 0)),
            pl.BlockSpec((r, d2p), lambda j: (nb + j, 0)),
            pl.BlockSpec((r, d2p), lambda j: (j, 0)),
            pl.BlockSpec((1, d_out), lambda j: (0, 0)),
        ],
        out_specs=pl.BlockSpec((r, d_out), lambda j: (j, 0)),
        out_shape=jax.ShapeDtypeStruct((npad, d_out), F32),
    )(dis, s2_flat, s2_flat, hs2p, b2.reshape(1, d_out))

    return out[:n]


# speedup vs baseline: 11.3892x; 11.3892x over previous
import functools

import jax
import jax.numpy as jnp
from jax import lax
from jax.experimental import pallas as pl
from jax.experimental.pallas import tpu as pltpu
from jax.experimental.pallas import tpu_sc as plsc

F32 = jnp.float32
I32 = jnp.int32

NC = 2
NS = 16
L = 16
CH = 128
ZR = 112


def _mesh():
    return plsc.VectorSubcoreMesh(core_axis_name="c", subcore_axis_name="s")


def _deg_body(npad, et, dst_hbm, ew_hbm, out_hbm, part, dstv, ewv):
    c = lax.axis_index("c")
    s = lax.axis_index("s")
    w = c * NS + s
    z16 = jnp.zeros((L,), F32)

    def zero(i, _):
        part[pl.ds(i * L, L)] = z16
        return 0

    lax.fori_loop(0, npad // L, zero, 0)

    base0 = w * et

    def chunk(t, _):
        b = base0 + t * CH
        pltpu.sync_copy(dst_hbm.at[pl.ds(b, CH)], dstv)
        pltpu.sync_copy(ew_hbm.at[pl.ds(b, CH)], ewv)
        for g in range(CH // L):
            d16 = dstv[pl.ds(g * L, L)]
            e16 = ewv[pl.ds(g * L, L)]
            plsc.addupdate_scatter(part, [d16], e16)
        return 0

    lax.fori_loop(0, et // CH, chunk, 0)
    pltpu.sync_copy(part, out_hbm.at[w])


def _make_deg(npad, epad):
    et = epad // (NC * NS)
    return pl.kernel(
        functools.partial(_deg_body, npad, et),
        out_type=jax.ShapeDtypeStruct((NC * NS, npad), F32),
        mesh=_mesh(),
        compiler_params=pltpu.CompilerParams(needs_layout_passes=False),
        scratch_types=[
            pltpu.VMEM((npad,), F32),
            pltpu.VMEM((CH,), I32),
            pltpu.VMEM((CH,), F32),
        ],
    )


def _agg_body(npad, n_chunks, d, feature_split, src_hbm, dst_hbm, ew_hbm,
              v_hbm, out_hbm, srcv, dstv, idxv, ewv, rows, zbuf, acc, sem):
    c = lax.axis_index("c")
    s = lax.axis_index("s")
    tr = npad // NS
    nvec = d // L
    z16 = jnp.zeros((L,), F32)

    def zb(i, _):
        for k in range(nvec):
            zbuf[i, pl.ds(k * L, L)] = z16
        return 0

    lax.fori_loop(0, ZR, zb, 0)

    row0 = s * tr

    def za(j, _):
        pltpu.sync_copy(zbuf, acc.at[pl.ds(row0 + j * ZR, ZR)])
        return 0

    lax.fori_loop(0, tr // ZR, za, 0)
    plsc.subcore_barrier()

    if feature_split:
        base0 = s * (n_chunks * CH)
        goff = c * npad
    else:
        base0 = (c * NS + s) * (n_chunks * CH)
        goff = 0

    def chunk(t, _):
        b = base0 + t * CH
        pltpu.sync_copy(src_hbm.at[pl.ds(b, CH)], srcv)
        pltpu.sync_copy(dst_hbm.at[pl.ds(b, CH)], dstv)
        pltpu.sync_copy(ew_hbm.at[pl.ds(b, CH)], ewv)
        for g in range(CH // L):
            idxv[pl.ds(g * L, L)] = srcv[pl.ds(g * L, L)] + goff
        pltpu.async_copy(v_hbm.at[idxv], rows, sem).wait()

        def scale(g, _):
            e16 = ewv[pl.ds(g * L, L)]
            for e in range(L):
                e2 = g * L + e
                sc = e16[e]
                for k in range(nvec):
                    rows[e2, pl.ds(k * L, L)] = rows[e2, pl.ds(k * L, L)] * sc
            return 0

        lax.fori_loop(0, CH // L, scale, 0)
        pltpu.sync_copy(rows, acc.at[dstv], add=True)
        return 0

    lax.fori_loop(0, n_chunks, chunk, 0)
    plsc.subcore_barrier()
    pltpu.sync_copy(acc.at[pl.ds(row0, tr)],
                    out_hbm.at[pl.ds(c * npad + row0, tr)])


def _make_agg(npad, epad, d, feature_split):
    n_chunks = epad // ((NS if feature_split else NC * NS) * CH)
    return pl.kernel(
        functools.partial(_agg_body, npad, n_chunks, d, feature_split),
        out_type=jax.ShapeDtypeStruct((NC * npad, d), F32),
        mesh=_mesh(),
        compiler_params=pltpu.CompilerParams(
            needs_layout_passes=False, use_tc_tiling_on_sc=False),
        scratch_types=[
            pltpu.VMEM((CH,), I32),
            pltpu.VMEM((CH,), I32),
            pltpu.VMEM((CH,), I32),
            pltpu.VMEM((CH,), F32),
            pltpu.VMEM((CH, d), F32),
            pltpu.VMEM((ZR, d), F32),
            pltpu.VMEM_SHARED((npad, d), F32),
            pltpu.SemaphoreType.DMA,
        ],
    )


def _k2_body(parts_ref, x_ref, w1_ref, xs_ref, dis_ref):
    parts = parts_ref[...]
    ones = jnp.ones((parts.shape[0], 1), F32)
    deg = 1.0 + lax.dot_general(parts, ones, (((0,), (0,)), ((), ())),
                                preferred_element_type=F32)
    dis = lax.rsqrt(deg)
    xw = jnp.dot(x_ref[...], w1_ref[...], preferred_element_type=F32,
                 precision=lax.Precision.HIGHEST)
    half = xw.shape[1] // 2
    xs_ref[0] = dis * xw[:, :half]
    xs_ref[1] = dis * xw[:, half:]
    dis_ref[...] = dis


def _k4_body(dis_ref, s0_ref, s1_ref, xs0_ref, xs1_ref, b1_ref, w2_ref,
             out_ref):
    dis = dis_ref[...]
    b1 = b1_ref[...]
    w2 = w2_ref[...]
    half = b1.shape[1] // 2
    h0 = jax.nn.relu(dis * (s0_ref[...] + xs0_ref[...]) + b1[:, :half])
    h1 = jax.nn.relu(dis * (s1_ref[...] + xs1_ref[...]) + b1[:, half:])
    hw2 = (jnp.dot(h0, w2[:half, :], preferred_element_type=F32,
                   precision=lax.Precision.HIGHEST)
           + jnp.dot(h1, w2[half:, :], preferred_element_type=F32,
                     precision=lax.Precision.HIGHEST))
    hs2 = dis * hw2
    pad = out_ref.shape[1] - hs2.shape[1]
    out_ref[...] = jnp.concatenate(
        [hs2, jnp.zeros((hs2.shape[0], pad), F32)], axis=1)


def _k6_body(d_out, dis_ref, s2a_ref, s2b_ref, hs_ref, b2_ref, out_ref):
    t = s2a_ref[...] + s2b_ref[...] + hs_ref[...]
    o = dis_ref[...] * t[:, :d_out] + b2_ref[...]
    m = jnp.max(o, axis=1, keepdims=True)
    lse = m + jnp.log(jnp.sum(jnp.exp(o - m), axis=1, keepdims=True))
    out_ref[...] = o - lse


def kernel(x, edge_index, edge_weight, W1, b1, W2, b2):
    n, d_in = x.shape
    e = edge_index.shape[1]
    d_hid = W1.shape[1]
    d_out = W2.shape[1]
    half = d_hid // 2
    d2p = 16

    r = 512
    npad = -(-n // r) * r
    nb = npad // r
    epad = -(-e // (NC * NS * CH)) * (NC * NS * CH)

    src = jnp.pad(edge_index[0], (0, epad - e))
    dst = jnp.pad(edge_index[1], (0, epad - e))
    ew = jnp.pad(edge_weight, (0, epad - e))
    xp = jnp.pad(x, ((0, npad - n), (0, 0)))

    parts = _make_deg(npad, epad)(dst, ew)

    xs3, dis = pl.pallas_call(
        _k2_body,
        grid=(nb,),
        in_specs=[
            pl.BlockSpec((NC * NS, r), lambda j: (0, j)),
            pl.BlockSpec((r, d_in), lambda j: (j, 0)),
            pl.BlockSpec((d_in, d_hid), lambda j: (0, 0)),
        ],
        out_specs=[
            pl.BlockSpec((NC, r, half), lambda j: (0, j, 0)),
            pl.BlockSpec((r, 1), lambda j: (j, 0)),
        ],
        out_shape=[
            jax.ShapeDtypeStruct((NC, npad, half), F32),
            jax.ShapeDtypeStruct((npad, 1), F32),
        ],
    )(parts, xp, W1)
    xs_flat = xs3.reshape(NC * npad, half)

    s1_flat = _make_agg(npad, epad, half, True)(src, dst, ew, xs_flat)

    hs2p = pl.pallas_call(
        _k4_body,
        grid=(nb,),
        in_specs=[
            pl.BlockSpec((r, 1), lambda j: (j, 0)),
            pl.BlockSpec((r, half), lambda j: (j, 0)),
            pl.BlockSpec((r, half), lambda j: (nb + j, 0)),
            pl.BlockSpec((r, half), lambda j: (j, 0)),
            pl.BlockSpec((r, half), lambda j: (nb + j, 0)),
            pl.BlockSpec((1, d_hid), lambda j: (0, 0)),
            pl.BlockSpec((d_hid, d_out), lambda j: (0, 0)),
        ],
        out_specs=pl.BlockSpec((r, d2p), lambda j: (j, 0)),
        out_shape=jax.ShapeDtypeStruct((npad, d2p), F32),
    )(dis, s1_flat, s1_flat, xs_flat, xs_flat, b1.reshape(1, d_hid), W2)

    s2_flat = _make_agg(npad, epad, d2p, False)(src, dst, ew, hs2p)

    out = pl.pallas_call(
        functools.partial(_k6_body, d_out),
        grid=(nb,),
        in_specs=[
            pl.BlockSpec((r, 1), lambda j: (j, 0)),
            pl.BlockSpec((r, d2p), lambda j: (j, 0)),
            pl.BlockSpec((r, d2p), lambda j: (nb + j, 0)),
            pl.BlockSpec((r, d2p), lambda j: (j, 0)),
            pl.BlockSpec((1, d_out), lambda j: (0, 0)),
        ],
        out_specs=pl.BlockSpec((r, d_out), lambda j: (j, 0)),
        out_shape=jax.ShapeDtypeStruct((npad, d_out), F32),
    )(dis, s2_flat, s2_flat, hs2p, b2.reshape(1, d_out))

    return out[:n]


# speedup vs baseline: 19.7150x; 1.7310x over previous
import functools

import jax
import jax.numpy as jnp
from jax import lax
from jax.experimental import pallas as pl
from jax.experimental.pallas import tpu as pltpu
from jax.experimental.pallas import tpu_sc as plsc

F32 = jnp.float32
I32 = jnp.int32

NC = 2
NS = 16
L = 16
CH = 128
ZR = 112


def _mesh():
    return plsc.VectorSubcoreMesh(core_axis_name="c", subcore_axis_name="s")


def _deg_body(npad, et, dst_hbm, ew_hbm, out_hbm, part, dstb, ewb):
    c = lax.axis_index("c")
    s = lax.axis_index("s")
    w = c * NS + s
    z16 = jnp.zeros((L,), F32)

    def zero(i, _):
        part[pl.ds(i * L, L)] = z16
        return 0

    lax.fori_loop(0, npad // L, zero, 0)

    base0 = w * et
    pltpu.sync_copy(dst_hbm.at[pl.ds(base0, et)], dstb)
    pltpu.sync_copy(ew_hbm.at[pl.ds(base0, et)], ewb)

    def group(g, _):
        d16 = dstb[pl.ds(g * L, L)]
        e16 = ewb[pl.ds(g * L, L)]
        plsc.addupdate_scatter(part, [d16], e16)
        return 0

    lax.fori_loop(0, et // L, group, 0)
    pltpu.sync_copy(part, out_hbm.at[w])


def _make_deg(npad, epad):
    et = epad // (NC * NS)
    return pl.kernel(
        functools.partial(_deg_body, npad, et),
        out_type=jax.ShapeDtypeStruct((NC * NS, npad), F32),
        mesh=_mesh(),
        compiler_params=pltpu.CompilerParams(needs_layout_passes=False),
        scratch_types=[
            pltpu.VMEM((npad,), F32),
            pltpu.VMEM((et,), I32),
            pltpu.VMEM((et,), F32),
        ],
    )


SB = 512
CPS = SB // CH


def _agg_body(npad, et, d, feature_split, src_hbm, dst_hbm, ew_hbm,
              v_hbm, out_hbm, srcb, dstb, ewb, idxg, idxs, rows, zbuf, acc,
              sg0, sg1, ss0, ss1, se):
    c = lax.axis_index("c")
    s = lax.axis_index("s")
    tr = npad // NS
    nvec = d // L
    n_super = et // SB
    z16 = jnp.zeros((L,), F32)
    zi16 = jnp.zeros((L,), I32)
    sg = (sg0, sg1)
    ss = (ss0, ss1)

    def zb(i, _):
        for k in range(nvec):
            zbuf[i, pl.ds(k * L, L)] = z16
        return 0

    lax.fori_loop(0, ZR, zb, 0)

    row0 = s * tr

    def za(j, _):
        pltpu.sync_copy(zbuf, acc.at[pl.ds(row0 + j * ZR, ZR)])
        return 0

    lax.fori_loop(0, tr // ZR, za, 0)
    plsc.subcore_barrier()

    if feature_split:
        tile_base = s * et
        goff = c * npad
    else:
        tile_base = (c * NS + s) * et
        goff = 0

    def load_edges(sc_i):
        eo = lax.rem(sc_i, 2) * SB
        b = tile_base + sc_i * SB
        pltpu.sync_copy(src_hbm.at[pl.ds(b, SB)], srcb.at[pl.ds(eo, SB)])
        pltpu.sync_copy(dst_hbm.at[pl.ds(b, SB)], dstb.at[pl.ds(eo, SB)])
        pltpu.sync_copy(ew_hbm.at[pl.ds(b, SB)], ewb.at[pl.ds(eo, SB)])

    def build_idx(p, off):
        for g in range(CH // L):
            sl = pl.ds(off + g * L, L)
            idxg[p, pl.ds(g * L, L)] = srcb[sl] + goff
            idxs[p, pl.ds(g * L, L)] = dstb[sl]

    def issue_gather(p):
        pltpu.async_copy(v_hbm.at[idxg.at[p]], rows.at[p], sg[p])

    def wait_gather(p):
        pltpu.make_async_copy(v_hbm.at[idxg.at[p]], rows.at[p], sg[p]).wait()

    def issue_scatter(p):
        pltpu.async_copy(rows.at[p], acc.at[idxs.at[p]], ss[p], add=True)

    def wait_scatter(p):
        pltpu.make_async_copy(rows.at[p], acc.at[idxs.at[p]], ss[p]).wait()

    def scale(p, off):
        def body(g, _):
            e16 = ewb[pl.ds(off + g * L, L)]
            for e in range(L):
                sc = e16[e]
                for k in range(nvec):
                    rows[p, g * L + e, pl.ds(k * L, L)] = (
                        rows[p, g * L + e, pl.ds(k * L, L)] * sc)
            return 0

        lax.fori_loop(0, CH // L, body, 0)

    for g in range(CH // L):
        idxs[1, pl.ds(g * L, L)] = zi16

    def zr(i, _):
        for k in range(nvec):
            rows[1, i, pl.ds(k * L, L)] = z16
        return 0

    lax.fori_loop(0, CH, zr, 0)
    issue_scatter(1)

    load_edges(0)
    build_idx(0, 0)
    issue_gather(0)

    def superchunk(sc_i, _):
        eo_cur = lax.rem(sc_i, 2) * SB
        eo_next = SB - eo_cur
        load_edges(sc_i + 1)
        for j in range(CPS):
            p = j % 2
            wait_gather(p)
            scale(p, eo_cur + j * CH)
            wait_scatter(1 - p)
            if j < CPS - 1:
                build_idx(1 - p, eo_cur + (j + 1) * CH)
            else:
                build_idx(1 - p, eo_next)
            issue_gather(1 - p)
            issue_scatter(p)
        return 0

    lax.fori_loop(0, n_super, superchunk, 0)
    wait_gather(0)
    wait_scatter(1)

    plsc.subcore_barrier()
    pltpu.sync_copy(acc.at[pl.ds(row0, tr)],
                    out_hbm.at[pl.ds(c * npad + row0, tr)])


def _make_agg(npad, epad, d, feature_split):
    et = epad // ((NS if feature_split else NC * NS))
    return pl.kernel(
        functools.partial(_agg_body, npad, et, d, feature_split),
        out_type=jax.ShapeDtypeStruct((NC * npad, d), F32),
        mesh=_mesh(),
        compiler_params=pltpu.CompilerParams(
            needs_layout_passes=False, use_tc_tiling_on_sc=False),
        scratch_types=[
            pltpu.VMEM((2 * SB,), I32),
            pltpu.VMEM((2 * SB,), I32),
            pltpu.VMEM((2 * SB,), F32),
            pltpu.VMEM((2, CH), I32),
            pltpu.VMEM((2, CH), I32),
            pltpu.VMEM((2, CH, d), F32),
            pltpu.VMEM((ZR, d), F32),
            pltpu.VMEM_SHARED((npad, d), F32),
            pltpu.SemaphoreType.DMA,
            pltpu.SemaphoreType.DMA,
            pltpu.SemaphoreType.DMA,
            pltpu.SemaphoreType.DMA,
            pltpu.SemaphoreType.DMA,
        ],
    )


def _k2_body(parts_ref, x_ref, w1_ref, xs_ref, dis_ref):
    parts = parts_ref[...]
    ones = jnp.ones((parts.shape[0], 1), F32)
    deg = 1.0 + lax.dot_general(parts, ones, (((0,), (0,)), ((), ())),
                                preferred_element_type=F32)
    dis = lax.rsqrt(deg)
    xw = jnp.dot(x_ref[...], w1_ref[...], preferred_element_type=F32,
                 precision=lax.Precision.HIGHEST)
    half = xw.shape[1] // 2
    xs_ref[0] = dis * xw[:, :half]
    xs_ref[1] = dis * xw[:, half:]
    dis_ref[...] = dis


def _k4_body(dis_ref, s0_ref, s1_ref, xs0_ref, xs1_ref, b1_ref, w2_ref,
             out_ref):
    dis = dis_ref[...]
    b1 = b1_ref[...]
    w2 = w2_ref[...]
    half = b1.shape[1] // 2
    h0 = jax.nn.relu(dis * (s0_ref[...] + xs0_ref[...]) + b1[:, :half])
    h1 = jax.nn.relu(dis * (s1_ref[...] + xs1_ref[...]) + b1[:, half:])
    hw2 = (jnp.dot(h0, w2[:half, :], preferred_element_type=F32,
                   precision=lax.Precision.HIGHEST)
           + jnp.dot(h1, w2[half:, :], preferred_element_type=F32,
                     precision=lax.Precision.HIGHEST))
    hs2 = dis * hw2
    pad = out_ref.shape[1] - hs2.shape[1]
    out_ref[...] = jnp.concatenate(
        [hs2, jnp.zeros((hs2.shape[0], pad), F32)], axis=1)


def _k6_body(d_out, dis_ref, s2a_ref, s2b_ref, hs_ref, b2_ref, out_ref):
    t = s2a_ref[...] + s2b_ref[...] + hs_ref[...]
    o = dis_ref[...] * t[:, :d_out] + b2_ref[...]
    m = jnp.max(o, axis=1, keepdims=True)
    lse = m + jnp.log(jnp.sum(jnp.exp(o - m), axis=1, keepdims=True))
    out_ref[...] = o - lse


def kernel(x, edge_index, edge_weight, W1, b1, W2, b2):
    n, d_in = x.shape
    e = edge_index.shape[1]
    d_hid = W1.shape[1]
    d_out = W2.shape[1]
    half = d_hid // 2
    d2p = 16

    r = 512
    npad = -(-n // r) * r
    nb = npad // r
    epad = -(-e // (NC * NS * CH)) * (NC * NS * CH)

    src = jnp.pad(edge_index[0], (0, epad + SB - e))
    dst = jnp.pad(edge_index[1], (0, epad + SB - e))
    ew = jnp.pad(edge_weight, (0, epad + SB - e))
    xp = jnp.pad(x, ((0, npad - n), (0, 0)))

    parts = _make_deg(npad, epad)(dst, ew)

    xs3, dis = pl.pallas_call(
        _k2_body,
        grid=(nb,),
        in_specs=[
            pl.BlockSpec((NC * NS, r), lambda j: (0, j)),
            pl.BlockSpec((r, d_in), lambda j: (j, 0)),
            pl.BlockSpec((d_in, d_hid), lambda j: (0, 0)),
        ],
        out_specs=[
            pl.BlockSpec((NC, r, half), lambda j: (0, j, 0)),
            pl.BlockSpec((r, 1), lambda j: (j, 0)),
        ],
        out_shape=[
            jax.ShapeDtypeStruct((NC, npad, half), F32),
            jax.ShapeDtypeStruct((npad, 1), F32),
        ],
    )(parts, xp, W1)
    xs_flat = xs3.reshape(NC * npad, half)

    s1_flat = _make_agg(npad, epad, half, True)(src, dst, ew, xs_flat)

    hs2p = pl.pallas_call(
        _k4_body,
        grid=(nb,),
        in_specs=[
            pl.BlockSpec((r, 1), lambda j: (j, 0)),
            pl.BlockSpec((r, half), lambda j: (j, 0)),
            pl.BlockSpec((r, half), lambda j: (nb + j, 0)),
            pl.BlockSpec((r, half), lambda j: (j, 0)),
            pl.BlockSpec((r, half), lambda j: (nb + j, 0)),
            pl.BlockSpec((1, d_hid), lambda j: (0, 0)),
            pl.BlockSpec((d_hid, d_out), lambda j: (0, 0)),
        ],
        out_specs=pl.BlockSpec((r, d2p), lambda j: (j, 0)),
        out_shape=jax.ShapeDtypeStruct((npad, d2p), F32),
    )(dis, s1_flat, s1_flat, xs_flat, xs_flat, b1.reshape(1, d_hid), W2)

    s2_flat = _make_agg(npad, epad, d2p, False)(src, dst, ew, hs2p)

    out = pl.pallas_call(
        functools.partial(_k6_body, d_out),
        grid=(nb,),
        in_specs=[
            pl.BlockSpec((r, 1), lambda j: (j, 0)),
            pl.BlockSpec((r, d2p), lambda j: (j, 0)),
            pl.BlockSpec((r, d2p), lambda j: (nb + j, 0)),
            pl.BlockSpec((r, d2p), lambda j: (j, 0)),
            pl.BlockSpec((1, d_out), lambda j: (0, 0)),
        ],
        out_specs=pl.BlockSpec((r, d_out), lambda j: (j, 0)),
        out_shape=jax.ShapeDtypeStruct((npad, d_out), F32),
    )(dis, s2_flat, s2_flat, hs2p, b2.reshape(1, d_out))

    return out[:n]


# speedup vs baseline: 25.0116x; 1.2687x over previous
import functools

import jax
import jax.numpy as jnp
from jax import lax
from jax.experimental import pallas as pl
from jax.experimental.pallas import tpu as pltpu
from jax.experimental.pallas import tpu_sc as plsc

F32 = jnp.float32
I32 = jnp.int32

NC = 2
NS = 16
L = 16
CH = 128
ZR = 56


def _mesh():
    return plsc.VectorSubcoreMesh(core_axis_name="c", subcore_axis_name="s")


def _deg_body(npad, et, dst_hbm, ew_hbm, out_hbm, part, dstb, ewb):
    c = lax.axis_index("c")
    s = lax.axis_index("s")
    w = c * NS + s
    z16 = jnp.zeros((L,), F32)

    def zero(i, _):
        part[pl.ds(i * L, L)] = z16
        return 0

    lax.fori_loop(0, npad // L, zero, 0)

    base0 = w * et
    pltpu.sync_copy(dst_hbm.at[pl.ds(base0, et)], dstb)
    pltpu.sync_copy(ew_hbm.at[pl.ds(base0, et)], ewb)

    def group(g, _):
        d16 = dstb[pl.ds(g * L, L)]
        e16 = ewb[pl.ds(g * L, L)]
        plsc.addupdate_scatter(part, [d16], e16)
        return 0

    lax.fori_loop(0, et // L, group, 0)
    pltpu.sync_copy(part, out_hbm.at[w])


def _make_deg(npad, epad):
    et = epad // (NC * NS)
    return pl.kernel(
        functools.partial(_deg_body, npad, et),
        out_type=jax.ShapeDtypeStruct((NC * NS, npad), F32),
        mesh=_mesh(),
        compiler_params=pltpu.CompilerParams(needs_layout_passes=False),
        scratch_types=[
            pltpu.VMEM((npad,), F32),
            pltpu.VMEM((et,), I32),
            pltpu.VMEM((et,), F32),
        ],
    )


SB = 512
CPS = SB // CH


def _agg_body(npad, et, d, feature_split, src_hbm, dst_hbm, ew_hbm,
              v_hbm, out_hbm, srcb, dstb, ewb, idxg, idxs, rows, zbuf, acc,
              sg0, sg1, sg2, sg3, ss0, ss1, ss2, ss3):
    c = lax.axis_index("c")
    s = lax.axis_index("s")
    tr = npad // NS
    nvec = d // L
    n_super = et // SB
    z16 = jnp.zeros((L,), F32)
    zi16 = jnp.zeros((L,), I32)
    sg = (sg0, sg1, sg2, sg3)
    ss = (ss0, ss1, ss2, ss3)

    def zb(i, _):
        for k in range(nvec):
            zbuf[i, pl.ds(k * L, L)] = z16
        return 0

    lax.fori_loop(0, ZR, zb, 0)

    row0 = s * tr

    def za(j, _):
        pltpu.sync_copy(zbuf, acc.at[pl.ds(row0 + j * ZR, ZR)])
        return 0

    lax.fori_loop(0, tr // ZR, za, 0)
    plsc.subcore_barrier()

    if feature_split:
        tile_base = s * et
        goff = c * npad
    else:
        tile_base = (c * NS + s) * et
        goff = 0

    def load_edges(sc_i):
        eo = lax.rem(sc_i, 2) * SB
        b = tile_base + sc_i * SB
        pltpu.sync_copy(src_hbm.at[pl.ds(b, SB)], srcb.at[pl.ds(eo, SB)])
        pltpu.sync_copy(dst_hbm.at[pl.ds(b, SB)], dstb.at[pl.ds(eo, SB)])
        pltpu.sync_copy(ew_hbm.at[pl.ds(b, SB)], ewb.at[pl.ds(eo, SB)])

    def build_idx(p, off):
        for g in range(CH // L):
            sl = pl.ds(off + g * L, L)
            idxg[p, pl.ds(g * L, L)] = srcb[sl] + goff
            idxs[p, pl.ds(g * L, L)] = dstb[sl]

    def issue_gather(p):
        pltpu.async_copy(v_hbm.at[idxg.at[p]], rows.at[p], sg[p])

    def wait_gather(p):
        pltpu.make_async_copy(v_hbm.at[idxg.at[p]], rows.at[p], sg[p]).wait()

    def issue_scatter(p):
        pltpu.async_copy(rows.at[p], acc.at[idxs.at[p]], ss[p], add=True)

    def wait_scatter(p):
        pltpu.make_async_copy(rows.at[p], acc.at[idxs.at[p]], ss[p]).wait()

    def scale(p, off):
        def body(g, _):
            e16 = ewb[pl.ds(off + g * L, L)]
            for e in range(L):
                sc = e16[e]
                for k in range(nvec):
                    rows[p, g * L + e, pl.ds(k * L, L)] = (
                        rows[p, g * L + e, pl.ds(k * L, L)] * sc)
            return 0

        lax.fori_loop(0, CH // L, body, 0)

    for q in (2, 3):
        for g in range(CH // L):
            idxs[q, pl.ds(g * L, L)] = zi16

        def zr(i, _):
            for k in range(nvec):
                rows[q, i, pl.ds(k * L, L)] = z16
            return 0

        lax.fori_loop(0, CH, zr, 0)
        issue_scatter(q)

    load_edges(0)
    build_idx(0, 0)
    issue_gather(0)
    build_idx(1, CH)
    issue_gather(1)

    def superchunk(sc_i, _):
        eo_cur = lax.rem(sc_i, 2) * SB
        eo_next = SB - eo_cur
        load_edges(sc_i + 1)
        for j in range(CPS):
            p = j
            q = (j + 2) % 4
            wait_gather(p)
            wait_scatter(q)
            if j < 2:
                build_idx(q, eo_cur + (j + 2) * CH)
            else:
                build_idx(q, eo_next + (j - 2) * CH)
            issue_gather(q)
            scale(p, eo_cur + j * CH)
            issue_scatter(p)
        return 0

    lax.fori_loop(0, n_super, superchunk, 0)
    wait_gather(0)
    wait_gather(1)
    wait_scatter(2)
    wait_scatter(3)

    plsc.subcore_barrier()
    pltpu.sync_copy(acc.at[pl.ds(row0, tr)],
                    out_hbm.at[pl.ds(c * npad + row0, tr)])


def _make_agg(npad, epad, d, feature_split):
    et = epad // ((NS if feature_split else NC * NS))
    return pl.kernel(
        functools.partial(_agg_body, npad, et, d, feature_split),
        out_type=jax.ShapeDtypeStruct((NC * npad, d), F32),
        mesh=_mesh(),
        compiler_params=pltpu.CompilerParams(
            needs_layout_passes=False, use_tc_tiling_on_sc=False),
        scratch_types=[
            pltpu.VMEM((2 * SB,), I32),
            pltpu.VMEM((2 * SB,), I32),
            pltpu.VMEM((2 * SB,), F32),
            pltpu.VMEM((4, CH), I32),
            pltpu.VMEM((4, CH), I32),
            pltpu.VMEM((4, CH, d), F32),
            pltpu.VMEM((ZR, d), F32),
            pltpu.VMEM_SHARED((npad, d), F32),
            pltpu.SemaphoreType.DMA,
            pltpu.SemaphoreType.DMA,
            pltpu.SemaphoreType.DMA,
            pltpu.SemaphoreType.DMA,
            pltpu.SemaphoreType.DMA,
            pltpu.SemaphoreType.DMA,
            pltpu.SemaphoreType.DMA,
            pltpu.SemaphoreType.DMA,
        ],
    )


def _k2_body(parts_ref, x_ref, w1_ref, xs_ref, dis_ref):
    parts = parts_ref[...]
    ones = jnp.ones((parts.shape[0], 1), F32)
    deg = 1.0 + lax.dot_general(parts, ones, (((0,), (0,)), ((), ())),
                                preferred_element_type=F32)
    dis = lax.rsqrt(deg)
    xw = jnp.dot(x_ref[...], w1_ref[...], preferred_element_type=F32,
                 precision=lax.Precision.HIGHEST)
    half = xw.shape[1] // 2
    xs_ref[0] = dis * xw[:, :half]
    xs_ref[1] = dis * xw[:, half:]
    dis_ref[...] = dis


def _k4_body(dis_ref, s0_ref, s1_ref, xs0_ref, xs1_ref, b1_ref, w2_ref,
             out_ref):
    dis = dis_ref[...]
    b1 = b1_ref[...]
    w2 = w2_ref[...]
    half = b1.shape[1] // 2
    h0 = jax.nn.relu(dis * (s0_ref[...] + xs0_ref[...]) + b1[:, :half])
    h1 = jax.nn.relu(dis * (s1_ref[...] + xs1_ref[...]) + b1[:, half:])
    hw2 = (jnp.dot(h0, w2[:half, :], preferred_element_type=F32,
                   precision=lax.Precision.HIGHEST)
           + jnp.dot(h1, w2[half:, :], preferred_element_type=F32,
                     precision=lax.Precision.HIGHEST))
    hs2 = dis * hw2
    pad = out_ref.shape[1] - hs2.shape[1]
    out_ref[...] = jnp.concatenate(
        [hs2, jnp.zeros((hs2.shape[0], pad), F32)], axis=1)


def _k6_body(d_out, dis_ref, s2a_ref, s2b_ref, hs_ref, b2_ref, out_ref):
    t = s2a_ref[...] + s2b_ref[...] + hs_ref[...]
    o = dis_ref[...] * t[:, :d_out] + b2_ref[...]
    m = jnp.max(o, axis=1, keepdims=True)
    lse = m + jnp.log(jnp.sum(jnp.exp(o - m), axis=1, keepdims=True))
    out_ref[...] = o - lse


def kernel(x, edge_index, edge_weight, W1, b1, W2, b2):
    n, d_in = x.shape
    e = edge_index.shape[1]
    d_hid = W1.shape[1]
    d_out = W2.shape[1]
    half = d_hid // 2
    d2p = 16

    r = 512
    npad = -(-n // r) * r
    nb = npad // r
    epad = -(-e // (NC * NS * CH)) * (NC * NS * CH)

    src = jnp.pad(edge_index[0], (0, epad + SB - e))
    dst = jnp.pad(edge_index[1], (0, epad + SB - e))
    ew = jnp.pad(edge_weight, (0, epad + SB - e))
    xp = jnp.pad(x, ((0, npad - n), (0, 0)))

    parts = _make_deg(npad, epad)(dst, ew)

    xs3, dis = pl.pallas_call(
        _k2_body,
        grid=(nb,),
        in_specs=[
            pl.BlockSpec((NC * NS, r), lambda j: (0, j)),
            pl.BlockSpec((r, d_in), lambda j: (j, 0)),
            pl.BlockSpec((d_in, d_hid), lambda j: (0, 0)),
        ],
        out_specs=[
            pl.BlockSpec((NC, r, half), lambda j: (0, j, 0)),
            pl.BlockSpec((r, 1), lambda j: (j, 0)),
        ],
        out_shape=[
            jax.ShapeDtypeStruct((NC, npad, half), F32),
            jax.ShapeDtypeStruct((npad, 1), F32),
        ],
    )(parts, xp, W1)
    xs_flat = xs3.reshape(NC * npad, half)

    s1_flat = _make_agg(npad, epad, half, True)(src, dst, ew, xs_flat)

    hs2p = pl.pallas_call(
        _k4_body,
        grid=(nb,),
        in_specs=[
            pl.BlockSpec((r, 1), lambda j: (j, 0)),
            pl.BlockSpec((r, half), lambda j: (j, 0)),
            pl.BlockSpec((r, half), lambda j: (nb + j, 0)),
            pl.BlockSpec((r, half), lambda j: (j, 0)),
            pl.BlockSpec((r, half), lambda j: (nb + j, 0)),
            pl.BlockSpec((1, d_hid), lambda j: (0, 0)),
            pl.BlockSpec((d_hid, d_out), lambda j: (0, 0)),
        ],
        out_specs=pl.BlockSpec((r, d2p), lambda j: (j, 0)),
        out_shape=jax.ShapeDtypeStruct((npad, d2p), F32),
    )(dis, s1_flat, s1_flat, xs_flat, xs_flat, b1.reshape(1, d_hid), W2)

    s2_flat = _make_agg(npad, epad, d2p, False)(src, dst, ew, hs2p)

    out = pl.pallas_call(
        functools.partial(_k6_body, d_out),
        grid=(nb,),
        in_specs=[
            pl.BlockSpec((r, 1), lambda j: (j, 0)),
            pl.BlockSpec((r, d2p), lambda j: (j, 0)),
            pl.BlockSpec((r, d2p), lambda j: (nb + j, 0)),
            pl.BlockSpec((r, d2p), lambda j: (j, 0)),
            pl.BlockSpec((1, d_out), lambda j: (0, 0)),
        ],
        out_specs=pl.BlockSpec((r, d_out), lambda j: (j, 0)),
        out_shape=jax.ShapeDtypeStruct((npad, d_out), F32),
    )(dis, s2_flat, s2_flat, hs2p, b2.reshape(1, d_out))

    return out[:n]


# speedup vs baseline: 29.4109x; 1.1759x over previous
import functools

import jax
import jax.numpy as jnp
from jax import lax
from jax.experimental import pallas as pl
from jax.experimental.pallas import tpu as pltpu
from jax.experimental.pallas import tpu_sc as plsc

F32 = jnp.float32
I32 = jnp.int32

NC = 2
NS = 16
L = 16
CH = 128
ZR = 56


def _mesh():
    return plsc.VectorSubcoreMesh(core_axis_name="c", subcore_axis_name="s")


def _deg_body(npad, et, dst_hbm, ew_hbm, out_hbm, part, dstb, ewb):
    c = lax.axis_index("c")
    s = lax.axis_index("s")
    w = c * NS + s
    z16 = jnp.zeros((L,), F32)

    def zero(i, _):
        part[pl.ds(i * L, L)] = z16
        return 0

    lax.fori_loop(0, npad // L, zero, 0)

    base0 = w * et
    pltpu.sync_copy(dst_hbm.at[pl.ds(base0, et)], dstb)
    pltpu.sync_copy(ew_hbm.at[pl.ds(base0, et)], ewb)

    def group(g, _):
        d16 = dstb[pl.ds(g * L, L)]
        e16 = ewb[pl.ds(g * L, L)]
        plsc.addupdate_scatter(part, [d16], e16)
        return 0

    lax.fori_loop(0, et // L, group, 0)
    pltpu.sync_copy(part, out_hbm.at[w])


def _make_deg(npad, epad):
    et = epad // (NC * NS)
    return pl.kernel(
        functools.partial(_deg_body, npad, et),
        out_type=jax.ShapeDtypeStruct((NC * NS, npad), F32),
        mesh=_mesh(),
        compiler_params=pltpu.CompilerParams(needs_layout_passes=False),
        scratch_types=[
            pltpu.VMEM((npad,), F32),
            pltpu.VMEM((et,), I32),
            pltpu.VMEM((et,), F32),
        ],
    )


SB = 512
CPS = SB // CH


def _agg_body(npad, et, d, feature_split, src_hbm, dst_hbm, ew_hbm,
              v_hbm, out_hbm, srcb, dstb, ewb, idxg, idxs, rows, zbuf, acc,
              sg0, sg1, sg2, sg3, ss0, ss1, ss2, ss3):
    c = lax.axis_index("c")
    s = lax.axis_index("s")
    tr = npad // NS
    nvec = d // L
    n_super = et // SB
    z16 = jnp.zeros((L,), F32)
    zi16 = jnp.zeros((L,), I32)
    sg = (sg0, sg1, sg2, sg3)
    ss = (ss0, ss1, ss2, ss3)

    def zb(i, _):
        for k in range(nvec):
            zbuf[i, pl.ds(k * L, L)] = z16
        return 0

    lax.fori_loop(0, ZR, zb, 0)

    row0 = s * tr

    def za(j, _):
        pltpu.sync_copy(zbuf, acc.at[pl.ds(row0 + j * ZR, ZR)])
        return 0

    lax.fori_loop(0, tr // ZR, za, 0)
    plsc.subcore_barrier()

    if feature_split:
        tile_base = s * et
        goff = c * npad
    else:
        tile_base = (c * NS + s) * et
        goff = 0

    def load_edges(sc_i):
        eo = lax.rem(sc_i, 2) * SB
        b = tile_base + sc_i * SB
        pltpu.sync_copy(src_hbm.at[pl.ds(b, SB)], srcb.at[pl.ds(eo, SB)])
        pltpu.sync_copy(dst_hbm.at[pl.ds(b, SB)], dstb.at[pl.ds(eo, SB)])
        pltpu.sync_copy(ew_hbm.at[pl.ds(b, SB)], ewb.at[pl.ds(eo, SB)])

    def build_idx(p, off):
        for g in range(CH // L):
            sl = pl.ds(off + g * L, L)
            idxg[p, pl.ds(g * L, L)] = srcb[sl] + goff
            idxs[p, pl.ds(g * L, L)] = dstb[sl]

    def issue_gather(p):
        pltpu.async_copy(v_hbm.at[idxg.at[p]], rows.at[p], sg[p])

    def wait_gather(p):
        pltpu.make_async_copy(v_hbm.at[idxg.at[p]], rows.at[p], sg[p]).wait()

    def issue_scatter(p):
        pltpu.async_copy(rows.at[p], acc.at[idxs.at[p]], ss[p], add=True)

    def wait_scatter(p):
        pltpu.make_async_copy(rows.at[p], acc.at[idxs.at[p]], ss[p]).wait()

    def scale(p, off):
        def body(g, _):
            e16 = ewb[pl.ds(off + g * L, L)]
            for e in range(L):
                sc = e16[e]
                for k in range(nvec):
                    rows[p, g * L + e, pl.ds(k * L, L)] = (
                        rows[p, g * L + e, pl.ds(k * L, L)] * sc)
            return 0

        lax.fori_loop(0, CH // L, body, 0)

    for q in (2, 3):
        for g in range(CH // L):
            idxs[q, pl.ds(g * L, L)] = zi16

        def zr(i, _):
            for k in range(nvec):
                rows[q, i, pl.ds(k * L, L)] = z16
            return 0

        lax.fori_loop(0, CH, zr, 0)
        issue_scatter(q)

    load_edges(0)
    build_idx(0, 0)
    issue_gather(0)
    build_idx(1, CH)
    issue_gather(1)

    def superchunk(sc_i, _):
        eo_cur = lax.rem(sc_i, 2) * SB
        eo_next = SB - eo_cur
        load_edges(sc_i + 1)
        for j in range(CPS):
            p = j
            q = (j + 2) % 4
            wait_gather(p)
            wait_scatter(q)
            if j < 2:
                build_idx(q, eo_cur + (j + 2) * CH)
            else:
                build_idx(q, eo_next + (j - 2) * CH)
            issue_gather(q)
            scale(p, eo_cur + j * CH)
            issue_scatter(p)
        return 0

    lax.fori_loop(0, n_super, superchunk, 0)
    wait_gather(0)
    wait_gather(1)
    wait_scatter(2)
    wait_scatter(3)

    plsc.subcore_barrier()
    pltpu.sync_copy(acc.at[pl.ds(row0, tr)],
                    out_hbm.at[pl.ds(c * npad + row0, tr)])


def _make_agg(npad, epad, d, feature_split):
    et = epad // ((NS if feature_split else NC * NS))
    return pl.kernel(
        functools.partial(_agg_body, npad, et, d, feature_split),
        out_type=jax.ShapeDtypeStruct((NC * npad, d), F32),
        mesh=_mesh(),
        compiler_params=pltpu.CompilerParams(
            needs_layout_passes=False, use_tc_tiling_on_sc=False),
        scratch_types=[
            pltpu.VMEM((2 * SB,), I32),
            pltpu.VMEM((2 * SB,), I32),
            pltpu.VMEM((2 * SB,), F32),
            pltpu.VMEM((4, CH), I32),
            pltpu.VMEM((4, CH), I32),
            pltpu.VMEM((4, CH, d), F32),
            pltpu.VMEM((ZR, d), F32),
            pltpu.VMEM_SHARED((npad, d), F32),
            pltpu.SemaphoreType.DMA,
            pltpu.SemaphoreType.DMA,
            pltpu.SemaphoreType.DMA,
            pltpu.SemaphoreType.DMA,
            pltpu.SemaphoreType.DMA,
            pltpu.SemaphoreType.DMA,
            pltpu.SemaphoreType.DMA,
            pltpu.SemaphoreType.DMA,
        ],
    )


def _k2_body(parts_ref, x_ref, w1_ref, xs_ref, dis_ref):
    parts = parts_ref[...]
    ones = jnp.ones((parts.shape[0], 1), F32)
    deg = 1.0 + lax.dot_general(parts, ones, (((0,), (0,)), ((), ())),
                                preferred_element_type=F32)
    dis = lax.rsqrt(deg)
    xw = jnp.dot(x_ref[...], w1_ref[...], preferred_element_type=F32,
                 precision=lax.Precision.HIGHEST)
    half = xw.shape[1] // 2
    xs_ref[0] = dis * xw[:, :half]
    xs_ref[1] = dis * xw[:, half:]
    dis_ref[...] = dis


def _k4_body(dis_ref, s0_ref, s1_ref, xs0_ref, xs1_ref, b1_ref, w2_ref,
             out_ref):
    dis = dis_ref[...]
    b1 = b1_ref[...]
    w2 = w2_ref[...]
    half = b1.shape[1] // 2
    h0 = jax.nn.relu(dis * (s0_ref[...] + xs0_ref[...]) + b1[:, :half])
    h1 = jax.nn.relu(dis * (s1_ref[...] + xs1_ref[...]) + b1[:, half:])
    hw2 = (jnp.dot(h0, w2[:half, :], preferred_element_type=F32,
                   precision=lax.Precision.HIGHEST)
           + jnp.dot(h1, w2[half:, :], preferred_element_type=F32,
                     precision=lax.Precision.HIGHEST))
    hs2 = dis * hw2
    pad = out_ref.shape[1] - hs2.shape[1]
    out_ref[...] = jnp.concatenate(
        [hs2, jnp.zeros((hs2.shape[0], pad), F32)], axis=1)


def _k6_body(d_out, dis_ref, s2a_ref, s2b_ref, hs_ref, b2_ref, out_ref):
    t = s2a_ref[...] + s2b_ref[...] + hs_ref[...]
    o = dis_ref[...] * t[:, :d_out] + b2_ref[...]
    m = jnp.max(o, axis=1, keepdims=True)
    lse = m + jnp.log(jnp.sum(jnp.exp(o - m), axis=1, keepdims=True))
    out_ref[...] = o - lse


def kernel(x, edge_index, edge_weight, W1, b1, W2, b2):
    n, d_in = x.shape
    e = edge_index.shape[1]
    d_hid = W1.shape[1]
    d_out = W2.shape[1]
    half = d_hid // 2
    d2p = 16

    npad = -(-n // 512) * 512
    r = 3584 if npad % 3584 == 0 else 512
    nb = npad // r
    epad = -(-e // (NC * NS * CH)) * (NC * NS * CH)

    ei = jnp.pad(edge_index, ((0, 0), (0, epad + SB - e)))
    src = ei[0]
    dst = ei[1]
    ew = jnp.pad(edge_weight, (0, epad + SB - e))
    xp = x

    parts = _make_deg(npad, epad)(dst, ew)

    xs3, dis = pl.pallas_call(
        _k2_body,
        grid=(nb,),
        in_specs=[
            pl.BlockSpec((NC * NS, r), lambda j: (0, j)),
            pl.BlockSpec((r, d_in), lambda j: (j, 0)),
            pl.BlockSpec((d_in, d_hid), lambda j: (0, 0)),
        ],
        out_specs=[
            pl.BlockSpec((NC, r, half), lambda j: (0, j, 0)),
            pl.BlockSpec((r, 1), lambda j: (j, 0)),
        ],
        out_shape=[
            jax.ShapeDtypeStruct((NC, npad, half), F32),
            jax.ShapeDtypeStruct((npad, 1), F32),
        ],
    )(parts, xp, W1)
    xs_flat = xs3.reshape(NC * npad, half)

    s1_flat = _make_agg(npad, epad, half, True)(src, dst, ew, xs_flat)

    hs2p = pl.pallas_call(
        _k4_body,
        grid=(nb,),
        in_specs=[
            pl.BlockSpec((r, 1), lambda j: (j, 0)),
            pl.BlockSpec((r, half), lambda j: (j, 0)),
            pl.BlockSpec((r, half), lambda j: (nb + j, 0)),
            pl.BlockSpec((r, half), lambda j: (j, 0)),
            pl.BlockSpec((r, half), lambda j: (nb + j, 0)),
            pl.BlockSpec((1, d_hid), lambda j: (0, 0)),
            pl.BlockSpec((d_hid, d_out), lambda j: (0, 0)),
        ],
        out_specs=pl.BlockSpec((r, d2p), lambda j: (j, 0)),
        out_shape=jax.ShapeDtypeStruct((npad, d2p), F32),
    )(dis, s1_flat, s1_flat, xs_flat, xs_flat, b1.reshape(1, d_hid), W2)

    s2_flat = _make_agg(npad, epad, d2p, False)(src, dst, ew, hs2p)

    out = pl.pallas_call(
        functools.partial(_k6_body, d_out),
        grid=(nb,),
        in_specs=[
            pl.BlockSpec((r, 1), lambda j: (j, 0)),
            pl.BlockSpec((r, d2p), lambda j: (j, 0)),
            pl.BlockSpec((r, d2p), lambda j: (nb + j, 0)),
            pl.BlockSpec((r, d2p), lambda j: (j, 0)),
            pl.BlockSpec((1, d_out), lambda j: (0, 0)),
        ],
        out_specs=pl.BlockSpec((r, d_out), lambda j: (j, 0)),
        out_shape=jax.ShapeDtypeStruct((n, d_out), F32),
    )(dis, s2_flat, s2_flat, hs2p, b2.reshape(1, d_out))

    return out
